# baseline (speedup 1.0000x reference)
import functools

import jax
import jax.numpy as jnp
from jax import lax
from jax.experimental import pallas as pl
from jax.experimental.pallas import tpu as pltpu
from jax.experimental.pallas import tpu_sc as plsc

T, K, E, D, F, L, R = 256, 2, 16, 1024, 1024, 4, 16


_NW = 16
_TPW = T // _NW

_sc_mesh = plsc.VectorSubcoreMesh(core_axis_name="c", subcore_axis_name="s")


@functools.partial(
    pl.kernel,
    mesh=_sc_mesh,
    out_type=jax.ShapeDtypeStruct((T * E,), jnp.float32),
    scratch_types=[
        pltpu.VMEM((2 * _TPW,), jnp.float32),
        pltpu.VMEM((2 * _TPW,), jnp.int32),
        pltpu.VMEM((_TPW * E,), jnp.float32),
    ],
)
def _routing_call(wmeta_hbm, imeta_hbm, c_hbm, wmeta_v, imeta_v, c_slab):
    nc = 2
    wid = lax.axis_index("s") * nc + lax.axis_index("c")

    @pl.when(wid < _NW)
    def _work():
        base = wid * _TPW
        pltpu.sync_copy(wmeta_hbm.at[pl.ds(2 * base, 2 * _TPW)], wmeta_v)
        pltpu.sync_copy(imeta_hbm.at[pl.ds(2 * base, 2 * _TPW)], imeta_v)

        eio = lax.iota(jnp.int32, E)
        w0 = wmeta_v[pl.ds(0, _TPW)]
        w1 = wmeta_v[pl.ds(_TPW, _TPW)]
        id0 = imeta_v[pl.ds(0, _TPW)]
        id1 = imeta_v[pl.ds(_TPW, _TPW)]
        ninv = 1.0 / (w0 + w1 + 1e-9)
        n0 = w0 * ninv
        n1 = w1 * ninv
        for i in range(_TPW):
            row = (jnp.where(eio == id0[i], n0[i], 0.0) +
                   jnp.where(eio == id1[i], n1[i], 0.0))
            c_slab[pl.ds(i * E, E)] = row

        pltpu.sync_copy(c_slab, c_hbm.at[pl.ds(base * E, _TPW * E)])


EPB = 2
GRID = E // EPB


def _moe_body(c_ref, li_ref, scal_ref, x_ref, wg_ref, wu_ref,
              wd_ref, la_ref, lb_ref, o_ref, xb_ref):
    step = pl.program_id(0)

    @pl.when(step == 0)
    def _init():
        o_ref[...] = jnp.zeros_like(o_ref)
        xb_ref[...] = x_ref[...].astype(jnp.bfloat16)

    xb = xb_ref[...]

    c = c_ref[...]
    eio = jax.lax.broadcasted_iota(jnp.int32, (T, E), 1)
    lio = jax.lax.broadcasted_iota(jnp.int32, (T, L), 1)
    sel = jnp.where(lio == li_ref[...], 1.0, 0.0) * scal_ref[...]

    acc = jnp.zeros((T, D), jnp.float32)
    for j in range(EPB):
        e = step * EPB + j
        cvec = jnp.sum(jnp.where(eio == e, c, 0.0),
                       axis=1, keepdims=True)

        wg = wg_ref[j].astype(jnp.bfloat16)
        wu = wu_ref[j].astype(jnp.bfloat16)
        g = jnp.dot(xb, wg, preferred_element_type=jnp.float32)
        u = jnp.dot(xb, wu, preferred_element_type=jnp.float32)
        h = (g / (1.0 + jnp.exp(-g))) * u
        wd = wd_ref[j].astype(jnp.bfloat16)
        o = jnp.dot(h.astype(jnp.bfloat16), wd,
                    preferred_element_type=jnp.float32)

        a4 = la_ref[:, j].reshape(L * R, D).astype(jnp.bfloat16)
        za = jax.lax.dot_general(xb, a4, (((1,), (1,)), ((), ())),
                                 preferred_element_type=jnp.float32)
        delta = o
        for l in range(L):
            zs = za[:, l * R:(l + 1) * R] * sel[:, l:l + 1]
            bl = lb_ref[l, j].astype(jnp.bfloat16)
            delta = delta + jax.lax.dot_general(
                zs.astype(jnp.bfloat16), bl, (((1,), (1,)), ((), ())),
                preferred_element_type=jnp.float32)

        acc = acc + cvec * delta

    o_ref[...] += acc


_moe_call = pl.pallas_call(
    _moe_body,
    grid=(GRID,),
    in_specs=[
        pl.BlockSpec((T, E), lambda s: (0, 0)),
        pl.BlockSpec((T, 1), lambda s: (0, 0)),
        pl.BlockSpec((1, L), lambda s: (0, 0)),
        pl.BlockSpec((T, D), lambda s: (0, 0)),
        pl.BlockSpec((EPB, D, F), lambda s: (s, 0, 0)),
        pl.BlockSpec((EPB, D, F), lambda s: (s, 0, 0)),
        pl.BlockSpec((EPB, F, D), lambda s: (s, 0, 0)),
        pl.BlockSpec((L, EPB, R, D), lambda s: (0, s, 0, 0)),
        pl.BlockSpec((L, EPB, D, R), lambda s: (0, s, 0, 0)),
    ],
    out_specs=pl.BlockSpec((T, D), lambda s: (0, 0)),
    out_shape=jax.ShapeDtypeStruct((T, D), jnp.float32),
    scratch_shapes=[pltpu.VMEM((T, D), jnp.bfloat16)],
    compiler_params=pltpu.CompilerParams(
        dimension_semantics=("arbitrary",),
        vmem_limit_bytes=63 * 1024 * 1024),
)


def kernel(hidden_states, topk_weights, w_gate, w_up, w_down, lora_a,
           lora_b, scalings, topk_ids, lora_indices):
    wpack = jnp.transpose(topk_weights.reshape(_NW, _TPW, K),
                          (0, 2, 1)).reshape(2 * T)
    ipack = jnp.transpose(topk_ids.reshape(_NW, _TPW, K),
                          (0, 2, 1)).reshape(2 * T)
    c = _routing_call(wpack, ipack).reshape(T, E)
    li = lora_indices.reshape(T, 1)
    scal = scalings.reshape(1, L)
    return _moe_call(c, li, scal, hidden_states,
                     w_gate, w_up, w_down, lora_a, lora_b)


# speedup vs baseline: 1.1563x; 1.1563x over previous
import jax
import jax.numpy as jnp
from jax.experimental import pallas as pl
from jax.experimental.pallas import tpu as pltpu

T, K, E, D, F, L, R = 256, 2, 16, 1024, 1024, 4, 16

EPB = 2
GRID = E // EPB


def _moe_body(tw_ref, ids_ref, li_ref, scal_ref, x_ref, wg_ref, wu_ref,
              wd_ref, la_ref, lb_ref, o_ref, xb_ref):
    step = pl.program_id(0)

    @pl.when(step == 0)
    def _init():
        o_ref[...] = jnp.zeros_like(o_ref)
        xb_ref[...] = x_ref[...].astype(jnp.bfloat16)

    xb = xb_ref[...]

    tw = tw_ref[...]
    twn = tw / (jnp.sum(tw, axis=1, keepdims=True) + 1e-9)
    lio = jax.lax.broadcasted_iota(jnp.int32, (T, L), 1)
    sel = jnp.where(lio == li_ref[...], 1.0, 0.0) * scal_ref[...]

    acc = jnp.zeros((T, D), jnp.float32)
    for j in range(EPB):
        e = step * EPB + j
        cvec = jnp.sum(jnp.where(ids_ref[...] == e, twn, 0.0),
                       axis=1, keepdims=True)

        wg = wg_ref[j].astype(jnp.bfloat16)
        wu = wu_ref[j].astype(jnp.bfloat16)
        g = jnp.dot(xb, wg, preferred_element_type=jnp.float32)
        u = jnp.dot(xb, wu, preferred_element_type=jnp.float32)
        h = (g / (1.0 + jnp.exp(-g))) * u
        wd = wd_ref[j].astype(jnp.bfloat16)
        o = jnp.dot(h.astype(jnp.bfloat16), wd,
                    preferred_element_type=jnp.float32)

        a4 = la_ref[:, j].reshape(L * R, D).astype(jnp.bfloat16)
        za = jax.lax.dot_general(xb, a4, (((1,), (1,)), ((), ())),
                                 preferred_element_type=jnp.float32)
        delta = o
        for l in range(L):
            zs = za[:, l * R:(l + 1) * R] * sel[:, l:l + 1]
            bl = lb_ref[l, j].astype(jnp.bfloat16)
            delta = delta + jax.lax.dot_general(
                zs.astype(jnp.bfloat16), bl, (((1,), (1,)), ((), ())),
                preferred_element_type=jnp.float32)

        acc = acc + cvec * delta

    o_ref[...] += acc


_moe_call = pl.pallas_call(
    _moe_body,
    grid=(GRID,),
    in_specs=[
        pl.BlockSpec((T, K), lambda s: (0, 0)),
        pl.BlockSpec((T, K), lambda s: (0, 0)),
        pl.BlockSpec((T, 1), lambda s: (0, 0)),
        pl.BlockSpec((1, L), lambda s: (0, 0)),
        pl.BlockSpec((T, D), lambda s: (0, 0)),
        pl.BlockSpec((EPB, D, F), lambda s: (s, 0, 0)),
        pl.BlockSpec((EPB, D, F), lambda s: (s, 0, 0)),
        pl.BlockSpec((EPB, F, D), lambda s: (s, 0, 0)),
        pl.BlockSpec((L, EPB, R, D), lambda s: (0, s, 0, 0)),
        pl.BlockSpec((L, EPB, D, R), lambda s: (0, s, 0, 0)),
    ],
    out_specs=pl.BlockSpec((T, D), lambda s: (0, 0)),
    out_shape=jax.ShapeDtypeStruct((T, D), jnp.float32),
    scratch_shapes=[pltpu.VMEM((T, D), jnp.bfloat16)],
    compiler_params=pltpu.CompilerParams(
        dimension_semantics=("arbitrary",),
        vmem_limit_bytes=63 * 1024 * 1024),
)


def kernel(hidden_states, topk_weights, w_gate, w_up, w_down, lora_a,
           lora_b, scalings, topk_ids, lora_indices):
    li = lora_indices.reshape(T, 1)
    scal = scalings.reshape(1, L)
    return _moe_call(topk_weights, topk_ids, li, scal, hidden_states,
                     w_gate, w_up, w_down, lora_a, lora_b)
